# Initial kernel scaffold; baseline (speedup 1.0000x reference)
import jax
import jax.numpy as jnp
from jax.experimental import pallas as pl


def kernel(x, edge_index, edge_types, norm, bases1, comp1, loop_w1, bias1, bases2, comp2, loop_w2, bias2):
    raise NotImplementedError("write your pallas kernel here")


# speedup vs baseline: 11.3168x; 11.3168x over previous
import functools

import jax
import jax.numpy as jnp
from jax import lax
from jax.experimental import pallas as pl
from jax.experimental.pallas import tpu as pltpu
from jax.experimental.pallas import tpu_sc as plsc

N = 10000
E = 320000
IN = 128
HID = 128
CLS = 16
R = 8
C = 4

NC = 2
NS = 16
LANES = 16
NW = NC * NS

K_EDGE = 80
E_TOT = E + N
EDGES_PER_TILE = -(-E_TOT // (NW * K_EDGE)) * K_EDGE
E_PAD = EDGES_PER_TILE * NW
N_PAD = 10240
N_PER_TILE = N_PAD // NS


def _proj_body(ck, d, pad, comp_ref, bases_ref, h_ref, out_ref):
    r = pl.program_id(0)
    w = comp_ref[r, 0] * bases_ref[0]
    for c in range(1, ck):
        w = w + comp_ref[r, c] * bases_ref[c]
    m = jnp.dot(h_ref[...], w, preferred_element_type=jnp.float32)
    if pad:
        m = jnp.concatenate(
            [m, jnp.zeros((m.shape[0], pad), jnp.float32)], axis=1)
    out_ref[0] = m


def _proj(comp_ext, bases_ext, h, bn=1000):
    rk, ck = comp_ext.shape
    d = bases_ext.shape[-1]
    nb = N // bn
    return pl.pallas_call(
        functools.partial(_proj_body, ck, d, HID - d),
        grid=(rk, nb),
        in_specs=[
            pl.BlockSpec(memory_space=pltpu.SMEM),
            pl.BlockSpec((ck, IN, d), lambda r, b: (0, 0, 0)),
            pl.BlockSpec((bn, IN), lambda r, b: (b, 0)),
        ],
        out_specs=pl.BlockSpec((1, bn, HID), lambda r, b: (r, b, 0)),
        out_shape=jax.ShapeDtypeStruct((rk, N, HID), jnp.float32),
    )(comp_ext, bases_ext, h)


def _combine_body(relu, dout, p_ref, b_ref, out_ref):
    acc = p_ref[0] + p_ref[1]
    acc = acc[:, :dout] + b_ref[...]
    out_ref[...] = jnp.maximum(acc, 0.0) if relu else acc


def _combine(partials, bias_row, relu, bn, nrows, dout):
    nb = nrows // bn
    d = partials.shape[-1]
    return pl.pallas_call(
        functools.partial(_combine_body, relu, dout),
        grid=(nb,),
        in_specs=[
            pl.BlockSpec((2, bn, d), lambda b: (0, b, 0)),
            pl.BlockSpec((1, dout), lambda b: (0, 0)),
        ],
        out_specs=pl.BlockSpec((bn, dout), lambda b: (b, 0)),
        out_shape=jax.ShapeDtypeStruct((nrows, dout), jnp.float32),
    )(partials, bias_row)


@functools.lru_cache(maxsize=None)
def _make_sc_scatter(d):
    n_chunks = EDGES_PER_TILE // K_EDGE
    mesh = plsc.VectorSubcoreMesh(
        core_axis_name="c", subcore_axis_name="s", num_cores=NC,
        num_subcores=NS)

    @functools.partial(
        pl.kernel,
        out_type=jax.ShapeDtypeStruct((NC, N_PAD, d), jnp.float32),
        mesh=mesh,
        scratch_types=[
            pltpu.VMEM((K_EDGE,), jnp.int32),
            pltpu.VMEM((K_EDGE,), jnp.int32),
            pltpu.VMEM((K_EDGE,), jnp.float32),
            pltpu.VMEM((K_EDGE, d), jnp.float32),
            pltpu.VMEM_SHARED((N_PAD, d), jnp.float32),
            pltpu.SemaphoreType.DMA,
        ],
    )
    def sc_scatter(table, gidx, dst, norm, zeros, out,
                   gidx_v, dst_v, norm_v, msg_v, agg_sh, sem):
        cid = lax.axis_index("c")
        sid = lax.axis_index("s")
        wid = cid * NS + sid
        pltpu.sync_copy(zeros.at[pl.ds(sid * N_PER_TILE, N_PER_TILE)],
                        agg_sh.at[pl.ds(sid * N_PER_TILE, N_PER_TILE)])
        plsc.subcore_barrier()

        base0 = wid * EDGES_PER_TILE

        def chunk(i, carry):
            base = base0 + i * K_EDGE
            pltpu.sync_copy(gidx.at[pl.ds(base, K_EDGE)], gidx_v)
            pltpu.sync_copy(dst.at[pl.ds(base, K_EDGE)], dst_v)
            pltpu.sync_copy(norm.at[pl.ds(base, K_EDGE)], norm_v)
            pltpu.async_copy(table.at[gidx_v], msg_v, sem).wait()

            def scale(g, c2):
                nv = norm_v[pl.ds(g * LANES, LANES)]
                for t in range(LANES):
                    nj = nv[t]
                    j = g * LANES + t
                    for q in range(d // LANES):
                        sl = pl.ds(q * LANES, LANES)
                        msg_v[j, sl] = msg_v[j, sl] * nj
                return c2

            lax.fori_loop(0, K_EDGE // LANES, scale, 0)
            pltpu.sync_copy(msg_v, agg_sh.at[dst_v], add=True)
            return carry

        lax.fori_loop(0, n_chunks, chunk, 0)
        plsc.subcore_barrier()
        pltpu.sync_copy(agg_sh.at[pl.ds(sid * N_PER_TILE, N_PER_TILE)],
                        out.at[cid, pl.ds(sid * N_PER_TILE, N_PER_TILE)])

    return sc_scatter


def kernel(x, edge_index, edge_types, norm, bases1, comp1, loop_w1, bias1,
           bases2, comp2, loop_w2, bias2):
    src = edge_index[0]
    dst = edge_index[1]
    ar = jnp.arange(N, dtype=jnp.int32)
    pad = E_PAD - E_TOT

    dst_all = jnp.concatenate(
        [dst, ar, jnp.zeros((pad,), jnp.int32)])
    norm_all = jnp.concatenate(
        [norm, jnp.ones((N,), jnp.float32), jnp.zeros((pad,), jnp.float32)])
    gidx = jnp.concatenate(
        [edge_types * N + src, R * N + ar, jnp.zeros((pad,), jnp.int32)])

    comp_ext1 = jnp.concatenate([
        jnp.concatenate([comp1, jnp.zeros((R, 1), jnp.float32)], axis=1),
        jnp.concatenate([jnp.zeros((1, C), jnp.float32),
                         jnp.ones((1, 1), jnp.float32)], axis=1),
    ], axis=0)
    comp_ext2 = jnp.concatenate([
        jnp.concatenate([comp2, jnp.zeros((R, 1), jnp.float32)], axis=1),
        jnp.concatenate([jnp.zeros((1, C), jnp.float32),
                         jnp.ones((1, 1), jnp.float32)], axis=1),
    ], axis=0)
    bases1_ext = jnp.concatenate([bases1, loop_w1[None]], axis=0)
    bases2_ext = jnp.concatenate([bases2, loop_w2[None]], axis=0)

    zeros128 = jnp.zeros((N_PAD, HID), jnp.float32)
    scat = _make_sc_scatter(HID)

    h1_tab = _proj(comp_ext1, bases1_ext, x)
    p1 = scat(h1_tab.reshape((R + 1) * N, HID), gidx, dst_all, norm_all,
              zeros128)
    h1 = _combine(p1, bias1.reshape(1, HID), relu=True, bn=1000, nrows=N,
                  dout=HID)

    h2_tab = _proj(comp_ext2, bases2_ext, h1)
    p2 = scat(h2_tab.reshape((R + 1) * N, HID), gidx, dst_all, norm_all,
              zeros128)
    return _combine(p2, bias2.reshape(1, CLS), relu=False, bn=1000, nrows=N,
                    dout=CLS)
